# baseline (speedup 1.0000x reference)
import functools

import jax
import jax.numpy as jnp
from jax import lax
from jax.experimental import pallas as pl
from jax.experimental.pallas import tpu as pltpu
from jax.experimental.pallas import tpu_sc as plsc

N_ATOMS = 10000
N_BONDS = 320000
MAX_NB = 32
HIDDEN = 128
DEPTH = 4
N_MOLS = 100
MOL_SIZE = 100

NAP = 10240
NC, NS = 2, 16
NW = NC * NS
AW = NAP // NW
BW = N_BONDS // NW
CH = 80
NCHUNK_A = AW // CH
NCHUNK_B = BW // CH

_MESH = plsc.VectorSubcoreMesh(core_axis_name="c", subcore_axis_name="s")
F32 = jnp.float32


def _wid():
    return lax.axis_index("s") * NC + lax.axis_index("c")


def _g0_body(m_hbm, a2bt_hbm, out_hbm, *rest, negate):
    idxs = rest[:NCHUNK_A]
    accs = rest[NCHUNK_A:2 * NCHUNK_A]
    sems = rest[2 * NCHUNK_A:3 * NCHUNK_A]
    base = _wid() * AW
    nidx = MAX_NB * AW
    for c in range(NCHUNK_A):
        off = pl.multiple_of(_wid() * nidx + c * (MAX_NB * CH), 8)
        pltpu.sync_copy(a2bt_hbm.at[pl.ds(off, MAX_NB * CH)], idxs[c])

    def _fire(k, add):
        for c in range(NCHUNK_A):
            off = pl.multiple_of(k * CH, 8)
            pltpu.async_copy(m_hbm.at[idxs[c].at[pl.ds(off, CH)]],
                             accs[c], sems[c], add=add)

    def _drain():
        for c in range(NCHUNK_A):
            pltpu.make_async_copy(m_hbm.at[idxs[c].at[pl.ds(0, CH)]],
                                  accs[c], sems[c]).wait()

    _fire(0, False)
    _drain()
    _fire(1, True)
    _fire(2, True)

    def body(k, carry):
        _fire(k, True)
        _drain()
        return carry

    lax.fori_loop(3, MAX_NB, body, 0)
    _drain()
    _drain()
    if negate:
        def nbody(r, carry):
            for c in range(NCHUNK_A):
                for j in range(HIDDEN // 16):
                    sl = pl.ds(j * 16, 16)
                    accs[c][r, sl] = -accs[c][r, sl]
            return carry

        lax.fori_loop(0, CH, nbody, 0)
    for c in range(NCHUNK_A):
        pltpu.sync_copy(accs[c], out_hbm.at[pl.ds(base + c * CH, CH), :])


@functools.partial(jax.jit, static_argnames="negate")
def _g0(m, a2bt, negate=False):
    return pl.kernel(
        functools.partial(_g0_body, negate=negate),
        out_type=jax.ShapeDtypeStruct((NAP, HIDDEN), F32),
        mesh=_MESH,
        scratch_types=(
            [pltpu.VMEM((MAX_NB * CH,), jnp.int32)] * NCHUNK_A
            + [pltpu.VMEM((CH, HIDDEN), F32)] * NCHUNK_A
            + [pltpu.SemaphoreType.DMA] * NCHUNK_A
        ),
    )(m, a2bt)


NSLOT = 8


def _g1_round(r, first, slots, refs):
    (ap_hbm, pn_hbm, b2a_hbm, b2revb_hbm, q_hbm,
     ia_v, ir_v, qbs, si, sg, so, base) = refs
    offs = {}
    for s in slots:
        g = pl.multiple_of(base + (r * NSLOT + s) * CH, CH)
        offs[s] = g
        pltpu.async_copy(b2a_hbm.at[pl.ds(g, CH)], ia_v.at[pl.ds(s * CH, CH)],
                         si[s])
        pltpu.async_copy(b2revb_hbm.at[pl.ds(g, CH)],
                         ir_v.at[pl.ds(s * CH, CH)], si[s])
    for s in slots:
        ia = ia_v.at[pl.ds(s * CH, CH)]
        ir = ir_v.at[pl.ds(s * CH, CH)]
        pltpu.make_async_copy(b2a_hbm.at[pl.ds(0, CH)], ia, si[s]).wait()
        pltpu.make_async_copy(b2a_hbm.at[pl.ds(0, CH)], ir, si[s]).wait()
        if not first:
            pltpu.make_async_copy(qbs[s], q_hbm.at[pl.ds(0, CH), :], so[s]).wait()
        pltpu.async_copy(ap_hbm.at[ia], qbs[s], sg[s])
    for s in slots:
        ia = ia_v.at[pl.ds(s * CH, CH)]
        ir = ir_v.at[pl.ds(s * CH, CH)]
        pltpu.make_async_copy(ap_hbm.at[ia], qbs[s], sg[s]).wait()
        pltpu.async_copy(pn_hbm.at[ir], qbs[s], sg[s], add=True)
    for s in slots:
        ir = ir_v.at[pl.ds(s * CH, CH)]
        pltpu.make_async_copy(pn_hbm.at[ir], qbs[s], sg[s]).wait()
        pltpu.async_copy(qbs[s], q_hbm.at[pl.ds(offs[s], CH), :], so[s])


def _g1_body(ap_hbm, pn_hbm, b2a_hbm, b2revb_hbm, q_hbm, ia_v, ir_v, *rest):
    qbs = rest[:NSLOT]
    si = rest[NSLOT:2 * NSLOT]
    sg = rest[2 * NSLOT:3 * NSLOT]
    so = rest[3 * NSLOT:4 * NSLOT]
    base = _wid() * BW
    refs = (ap_hbm, pn_hbm, b2a_hbm, b2revb_hbm, q_hbm,
            ia_v, ir_v, qbs, si, sg, so, base)
    nround = NCHUNK_B // NSLOT
    ntail = NCHUNK_B - nround * NSLOT
    _g1_round(0, True, range(NSLOT), refs)

    def body(r, carry):
        _g1_round(r, False, range(NSLOT), refs)
        return carry

    lax.fori_loop(1, nround, body, 0)
    _g1_round(nround, False, range(ntail), refs)
    for s in range(NSLOT):
        pltpu.make_async_copy(qbs[s], q_hbm.at[pl.ds(0, CH), :], so[s]).wait()


@jax.jit
def _g1(ap, pn, b2a, b2revb):
    return pl.kernel(
        _g1_body,
        out_type=jax.ShapeDtypeStruct((N_BONDS, HIDDEN), F32),
        mesh=_MESH,
        scratch_types=(
            [pltpu.VMEM((NSLOT * CH,), jnp.int32)] * 2
            + [pltpu.VMEM((CH, HIDDEN), F32)] * NSLOT
            + [pltpu.SemaphoreType.DMA] * (3 * NSLOT)
        ),
    )(ap, pn, b2a, b2revb)


RB = 2560
RBB = 12800


def _in_body(x_ref, wi_ref, whn_ref, pn_ref):
    inp = jnp.dot(x_ref[...], wi_ref[...], preferred_element_type=F32)
    pn_ref[...] = jnp.dot(jnp.maximum(inp, 0.0), whn_ref[...],
                          preferred_element_type=F32)


def _fuse_body(q_ref, fb_ref, wi_ref, whn_ref, pn_ref):
    inp = jnp.dot(fb_ref[...], wi_ref[...], preferred_element_type=F32)
    m = jnp.maximum(inp + q_ref[...], 0.0)
    pn_ref[...] = jnp.dot(m, whn_ref[...], preferred_element_type=F32)


def _fuse_last_body(q_ref, fb_ref, wi_ref, m_ref):
    inp = jnp.dot(fb_ref[...], wi_ref[...], preferred_element_type=F32)
    m_ref[...] = jnp.maximum(inp + q_ref[...], 0.0)


def _atom_body(fa_ref, am_ref, wo_ref, bo_ref, o_ref):
    acc = jnp.dot(fa_ref[...], wo_ref[:HIDDEN, :], preferred_element_type=F32)
    acc += jnp.dot(am_ref[...], wo_ref[HIDDEN:, :], preferred_element_type=F32)
    o_ref[...] = jnp.maximum(acc + bo_ref[...], 0.0)


def _row_spec(nrows, rb=RB):
    return pl.BlockSpec((rb, HIDDEN), lambda i: (i, 0))


def _w_spec(r=HIDDEN):
    return pl.BlockSpec((r, HIDDEN), lambda i: (0, 0))


@jax.jit
def _k_in(f_bonds, w_i, whn):
    n = N_BONDS // RBB
    sds = jax.ShapeDtypeStruct((N_BONDS, HIDDEN), F32)
    return pl.pallas_call(
        _in_body,
        grid=(n,),
        in_specs=[_row_spec(N_BONDS, RBB), _w_spec(), _w_spec()],
        out_specs=_row_spec(N_BONDS, RBB),
        out_shape=sds,
    )(f_bonds, w_i, whn)


@jax.jit
def _k_fuse(q, f_bonds, w_i, whn):
    n = N_BONDS // RBB
    sds = jax.ShapeDtypeStruct((N_BONDS, HIDDEN), F32)
    return pl.pallas_call(
        _fuse_body,
        grid=(n,),
        in_specs=[_row_spec(N_BONDS, RBB), _row_spec(N_BONDS, RBB),
                  _w_spec(), _w_spec()],
        out_specs=_row_spec(N_BONDS, RBB),
        out_shape=sds,
    )(q, f_bonds, w_i, whn)


@jax.jit
def _k_fuse_last(q, f_bonds, w_i):
    n = N_BONDS // RBB
    sds = jax.ShapeDtypeStruct((N_BONDS, HIDDEN), F32)
    return pl.pallas_call(
        _fuse_last_body,
        grid=(n,),
        in_specs=[_row_spec(N_BONDS, RBB), _row_spec(N_BONDS, RBB),
                  _w_spec()],
        out_specs=_row_spec(N_BONDS, RBB),
        out_shape=sds,
    )(q, f_bonds, w_i)


@jax.jit
def _k_atom(fa, am, wo, bo):
    n = NAP // RB
    return pl.pallas_call(
        _atom_body,
        grid=(n,),
        in_specs=[_row_spec(NAP), _row_spec(NAP), _w_spec(2 * HIDDEN),
                  pl.BlockSpec((1, HIDDEN), lambda i: (0, 0))],
        out_specs=_row_spec(NAP),
        out_shape=jax.ShapeDtypeStruct((NAP, HIDDEN), F32),
    )(fa, am, wo, bo)


MB = 4


def _att_core_body(h_ref, t_ref, z_ref):
    for i in range(MB):
        cur = h_ref[i]
        logits = lax.dot_general(t_ref[i], cur, (((1,), (1,)), ((), ())),
                                 preferred_element_type=F32)
        logits = logits - jnp.max(logits, axis=1, keepdims=True)
        e = jnp.exp(logits)
        att = e / jnp.sum(e, axis=1, keepdims=True)
        z_ref[i] = jnp.dot(att, cur, preferred_element_type=F32)


@jax.jit
def _k_att_core(h3, t3):
    spec = pl.BlockSpec((MB, MOL_SIZE, HIDDEN), lambda i: (i, 0, 0))
    return pl.pallas_call(
        _att_core_body,
        grid=(N_MOLS // MB,),
        in_specs=[spec, spec],
        out_specs=spec,
        out_shape=jax.ShapeDtypeStruct((N_MOLS, MOL_SIZE, HIDDEN), F32),
    )(h3, t3)


def _att_out_body(h_ref, z_ref, wb_ref, bb_ref, sz_ref, o_ref):
    for i in range(MB):
        ah = jnp.maximum(
            jnp.dot(z_ref[i], wb_ref[...], preferred_element_type=F32)
            + bb_ref[...], 0.0)
        o_ref[i] = (jnp.sum(h_ref[i] + ah, axis=0, keepdims=True)
                    / sz_ref[i, 0, 0])


@jax.jit
def _k_att_out(h3, z3, wb, bb, sz):
    spec = pl.BlockSpec((MB, MOL_SIZE, HIDDEN), lambda i: (i, 0, 0))
    return pl.pallas_call(
        _att_out_body,
        grid=(N_MOLS // MB,),
        in_specs=[
            spec,
            spec,
            _w_spec(),
            pl.BlockSpec((1, HIDDEN), lambda i: (0, 0)),
            pl.BlockSpec((MB, 1, 1), lambda i: (i, 0, 0)),
        ],
        out_specs=pl.BlockSpec((MB, 1, HIDDEN), lambda i: (i, 0, 0)),
        out_shape=jax.ShapeDtypeStruct((N_MOLS, 1, HIDDEN), F32),
    )(h3, z3, wb, bb, sz)


def _mm_body(x_ref, w_ref, o_ref):
    o_ref[...] = jnp.dot(x_ref[...], w_ref[...], preferred_element_type=F32)


@jax.jit
def _k_mm(x, w):
    n = NAP // RB
    return pl.pallas_call(
        _mm_body,
        grid=(n,),
        in_specs=[_row_spec(NAP), _w_spec()],
        out_specs=_row_spec(NAP),
        out_shape=jax.ShapeDtypeStruct((NAP, HIDDEN), F32),
    )(x, w)


def kernel(f_atoms, f_bonds, a2b, b2a, b2revb, a_scope,
           W_i, W_h, W_o, b_o, W_a, W_b, b_b):
    whn = -W_h
    a2bt = (jnp.pad(a2b, ((0, NAP - N_ATOMS), (0, 0)))
            .reshape(NW, NCHUNK_A, CH, MAX_NB).transpose(0, 1, 3, 2)
            .reshape(-1))

    pn = _k_in(f_bonds, W_i, whn)
    for t in range(DEPTH - 1):
        ap = _g0(pn, a2bt, negate=True)
        q = _g1(ap, pn, b2a, b2revb)
        if t == DEPTH - 2:
            m = _k_fuse_last(q, f_bonds, W_i)
        else:
            pn = _k_fuse(q, f_bonds, W_i, whn)
    amsg = _g0(m, a2bt)

    fa_p = jnp.pad(f_atoms, ((0, NAP - N_ATOMS), (0, 0)))
    ah = _k_atom(fa_p, amsg, W_o, b_o.reshape(1, HIDDEN))
    t_all = _k_mm(ah, W_a)
    h3 = ah[:N_ATOMS].reshape(N_MOLS, MOL_SIZE, HIDDEN)
    t3 = t_all[:N_ATOMS].reshape(N_MOLS, MOL_SIZE, HIDDEN)
    z3 = _k_att_core(h3, t3)
    sz = a_scope[:, 1].astype(F32).reshape(N_MOLS, 1, 1)
    out = _k_att_out(h3, z3, W_b, b_b.reshape(1, HIDDEN), sz)
    return out.reshape(N_MOLS, HIDDEN)


# speedup vs baseline: 1.1976x; 1.1976x over previous
import functools

import jax
import jax.numpy as jnp
from jax import lax
from jax.experimental import pallas as pl
from jax.experimental.pallas import tpu as pltpu
from jax.experimental.pallas import tpu_sc as plsc

N_ATOMS = 10000
N_BONDS = 320000
MAX_NB = 32
HIDDEN = 128
DEPTH = 4
N_MOLS = 100
MOL_SIZE = 100

NAP = 10240
NC, NS = 2, 16
NW = NC * NS
AW = NAP // NW
BW = N_BONDS // NW
CH = 80
NCHUNK_A = AW // CH
NCHUNK_B = BW // CH

_MESH = plsc.VectorSubcoreMesh(core_axis_name="c", subcore_axis_name="s")
F32 = jnp.float32


def _wid():
    return lax.axis_index("s") * NC + lax.axis_index("c")


def _g0_body(m_hbm, a2bt_hbm, out_hbm, *rest, negate):
    idxs = rest[:NCHUNK_A]
    accs = rest[NCHUNK_A:2 * NCHUNK_A]
    sems = rest[2 * NCHUNK_A:3 * NCHUNK_A]
    base = _wid() * AW
    nidx = MAX_NB * AW
    for c in range(NCHUNK_A):
        off = pl.multiple_of(_wid() * nidx + c * (MAX_NB * CH), 8)
        pltpu.sync_copy(a2bt_hbm.at[pl.ds(off, MAX_NB * CH)], idxs[c])

    def _fire(k, add):
        for c in range(NCHUNK_A):
            off = pl.multiple_of(k * CH, 8)
            pltpu.async_copy(m_hbm.at[idxs[c].at[pl.ds(off, CH)]],
                             accs[c], sems[c], add=add)

    def _drain():
        for c in range(NCHUNK_A):
            pltpu.make_async_copy(m_hbm.at[idxs[c].at[pl.ds(0, CH)]],
                                  accs[c], sems[c]).wait()

    _fire(0, False)
    _drain()
    _fire(1, True)
    _fire(2, True)

    def body(k, carry):
        _fire(k, True)
        _drain()
        return carry

    lax.fori_loop(3, MAX_NB, body, 0)
    _drain()
    _drain()
    if negate:
        def nbody(r, carry):
            for c in range(NCHUNK_A):
                for j in range(HIDDEN // 16):
                    sl = pl.ds(j * 16, 16)
                    accs[c][r, sl] = -accs[c][r, sl]
            return carry

        lax.fori_loop(0, CH, nbody, 0)
    for c in range(NCHUNK_A):
        pltpu.sync_copy(accs[c], out_hbm.at[pl.ds(base + c * CH, CH), :])


@functools.partial(jax.jit, static_argnames="negate")
def _g0(m, a2bt, negate=False):
    return pl.kernel(
        functools.partial(_g0_body, negate=negate),
        out_type=jax.ShapeDtypeStruct((NAP, HIDDEN), F32),
        mesh=_MESH,
        scratch_types=(
            [pltpu.VMEM((MAX_NB * CH,), jnp.int32)] * NCHUNK_A
            + [pltpu.VMEM((CH, HIDDEN), F32)] * NCHUNK_A
            + [pltpu.SemaphoreType.DMA] * NCHUNK_A
        ),
    )(m, a2bt)


NSLOT = 8


def _g1_round(r, first, slots, refs):
    (ap_hbm, pn_hbm, b2a_hbm, b2revb_hbm, q_hbm,
     ia_v, ir_v, qbs, si, sg, so, base) = refs
    offs = {}
    for s in slots:
        g = pl.multiple_of(base + (r * NSLOT + s) * CH, CH)
        offs[s] = g
        pltpu.async_copy(b2a_hbm.at[pl.ds(g, CH)], ia_v.at[pl.ds(s * CH, CH)],
                         si[s])
        pltpu.async_copy(b2revb_hbm.at[pl.ds(g, CH)],
                         ir_v.at[pl.ds(s * CH, CH)], si[s])
    for s in slots:
        ia = ia_v.at[pl.ds(s * CH, CH)]
        ir = ir_v.at[pl.ds(s * CH, CH)]
        pltpu.make_async_copy(b2a_hbm.at[pl.ds(0, CH)], ia, si[s]).wait()
        pltpu.make_async_copy(b2a_hbm.at[pl.ds(0, CH)], ir, si[s]).wait()
        if not first:
            pltpu.make_async_copy(qbs[s], q_hbm.at[pl.ds(0, CH), :], so[s]).wait()
        pltpu.async_copy(ap_hbm.at[ia], qbs[s], sg[s])
    for s in slots:
        ia = ia_v.at[pl.ds(s * CH, CH)]
        ir = ir_v.at[pl.ds(s * CH, CH)]
        pltpu.make_async_copy(ap_hbm.at[ia], qbs[s], sg[s]).wait()
        pltpu.async_copy(pn_hbm.at[ir], qbs[s], sg[s], add=True)
    for s in slots:
        ir = ir_v.at[pl.ds(s * CH, CH)]
        pltpu.make_async_copy(pn_hbm.at[ir], qbs[s], sg[s]).wait()
        pltpu.async_copy(qbs[s], q_hbm.at[pl.ds(offs[s], CH), :], so[s])


def _g1_body(ap_hbm, pn_hbm, b2a_hbm, b2revb_hbm, q_hbm, ia_v, ir_v, *rest):
    qbs = rest[:NSLOT]
    si = rest[NSLOT:2 * NSLOT]
    sg = rest[2 * NSLOT:3 * NSLOT]
    so = rest[3 * NSLOT:4 * NSLOT]
    base = _wid() * BW
    refs = (ap_hbm, pn_hbm, b2a_hbm, b2revb_hbm, q_hbm,
            ia_v, ir_v, qbs, si, sg, so, base)
    nround = NCHUNK_B // NSLOT
    ntail = NCHUNK_B - nround * NSLOT
    _g1_round(0, True, range(NSLOT), refs)

    def body(r, carry):
        _g1_round(r, False, range(NSLOT), refs)
        return carry

    lax.fori_loop(1, nround, body, 0)
    _g1_round(nround, False, range(ntail), refs)
    for s in range(NSLOT):
        pltpu.make_async_copy(qbs[s], q_hbm.at[pl.ds(0, CH), :], so[s]).wait()


@jax.jit
def _g1(ap, pn, b2a, b2revb):
    return pl.kernel(
        _g1_body,
        out_type=jax.ShapeDtypeStruct((N_BONDS, HIDDEN), F32),
        mesh=_MESH,
        scratch_types=(
            [pltpu.VMEM((NSLOT * CH,), jnp.int32)] * 2
            + [pltpu.VMEM((CH, HIDDEN), F32)] * NSLOT
            + [pltpu.SemaphoreType.DMA] * (3 * NSLOT)
        ),
    )(ap, pn, b2a, b2revb)


RB = 2560
RBB = 12800


def _in_body(x_ref, wi_ref, whn_ref, inp_ref, pn_ref):
    inp = jnp.dot(x_ref[...], wi_ref[...], preferred_element_type=F32)
    inp_ref[...] = inp
    pn_ref[...] = jnp.dot(jnp.maximum(inp, 0.0), whn_ref[...],
                          preferred_element_type=F32)


def _fuse_body(q_ref, inp_ref, whn_ref, pn_ref):
    m = jnp.maximum(inp_ref[...] + q_ref[...], 0.0)
    pn_ref[...] = jnp.dot(m, whn_ref[...], preferred_element_type=F32)


def _fuse_last_body(q_ref, inp_ref, m_ref):
    m_ref[...] = jnp.maximum(inp_ref[...] + q_ref[...], 0.0)


def _atom_body(fa_ref, am_ref, wo_ref, bo_ref, o_ref):
    acc = jnp.dot(fa_ref[...], wo_ref[:HIDDEN, :], preferred_element_type=F32)
    acc += jnp.dot(am_ref[...], wo_ref[HIDDEN:, :], preferred_element_type=F32)
    o_ref[...] = jnp.maximum(acc + bo_ref[...], 0.0)


def _row_spec(nrows, rb=RB):
    return pl.BlockSpec((rb, HIDDEN), lambda i: (i, 0))


def _w_spec(r=HIDDEN):
    return pl.BlockSpec((r, HIDDEN), lambda i: (0, 0))


@jax.jit
def _k_in(f_bonds, w_i, whn):
    n = N_BONDS // RBB
    sds = jax.ShapeDtypeStruct((N_BONDS, HIDDEN), F32)
    return pl.pallas_call(
        _in_body,
        grid=(n,),
        in_specs=[_row_spec(N_BONDS, RBB), _w_spec(), _w_spec()],
        out_specs=[_row_spec(N_BONDS, RBB)] * 2,
        out_shape=[sds, sds],
    )(f_bonds, w_i, whn)


@jax.jit
def _k_fuse(q, inputs, whn):
    n = N_BONDS // RBB
    sds = jax.ShapeDtypeStruct((N_BONDS, HIDDEN), F32)
    return pl.pallas_call(
        _fuse_body,
        grid=(n,),
        in_specs=[_row_spec(N_BONDS, RBB), _row_spec(N_BONDS, RBB), _w_spec()],
        out_specs=_row_spec(N_BONDS, RBB),
        out_shape=sds,
    )(q, inputs, whn)


@jax.jit
def _k_fuse_last(q, inputs):
    n = N_BONDS // RBB
    sds = jax.ShapeDtypeStruct((N_BONDS, HIDDEN), F32)
    return pl.pallas_call(
        _fuse_last_body,
        grid=(n,),
        in_specs=[_row_spec(N_BONDS, RBB), _row_spec(N_BONDS, RBB)],
        out_specs=_row_spec(N_BONDS, RBB),
        out_shape=sds,
    )(q, inputs)


@jax.jit
def _k_atom(fa, am, wo, bo):
    n = NAP // RB
    return pl.pallas_call(
        _atom_body,
        grid=(n,),
        in_specs=[_row_spec(NAP), _row_spec(NAP), _w_spec(2 * HIDDEN),
                  pl.BlockSpec((1, HIDDEN), lambda i: (0, 0))],
        out_specs=_row_spec(NAP),
        out_shape=jax.ShapeDtypeStruct((NAP, HIDDEN), F32),
    )(fa, am, wo, bo)


MB = 4


def _att_core_body(h_ref, t_ref, z_ref):
    for i in range(MB):
        cur = h_ref[i]
        logits = lax.dot_general(t_ref[i], cur, (((1,), (1,)), ((), ())),
                                 preferred_element_type=F32)
        logits = logits - jnp.max(logits, axis=1, keepdims=True)
        e = jnp.exp(logits)
        att = e / jnp.sum(e, axis=1, keepdims=True)
        z_ref[i] = jnp.dot(att, cur, preferred_element_type=F32)


@jax.jit
def _k_att_core(h3, t3):
    spec = pl.BlockSpec((MB, MOL_SIZE, HIDDEN), lambda i: (i, 0, 0))
    return pl.pallas_call(
        _att_core_body,
        grid=(N_MOLS // MB,),
        in_specs=[spec, spec],
        out_specs=spec,
        out_shape=jax.ShapeDtypeStruct((N_MOLS, MOL_SIZE, HIDDEN), F32),
    )(h3, t3)


def _att_out_body(h_ref, z_ref, wb_ref, bb_ref, sz_ref, o_ref):
    for i in range(MB):
        ah = jnp.maximum(
            jnp.dot(z_ref[i], wb_ref[...], preferred_element_type=F32)
            + bb_ref[...], 0.0)
        o_ref[i] = (jnp.sum(h_ref[i] + ah, axis=0, keepdims=True)
                    / sz_ref[i, 0, 0])


@jax.jit
def _k_att_out(h3, z3, wb, bb, sz):
    spec = pl.BlockSpec((MB, MOL_SIZE, HIDDEN), lambda i: (i, 0, 0))
    return pl.pallas_call(
        _att_out_body,
        grid=(N_MOLS // MB,),
        in_specs=[
            spec,
            spec,
            _w_spec(),
            pl.BlockSpec((1, HIDDEN), lambda i: (0, 0)),
            pl.BlockSpec((MB, 1, 1), lambda i: (i, 0, 0)),
        ],
        out_specs=pl.BlockSpec((MB, 1, HIDDEN), lambda i: (i, 0, 0)),
        out_shape=jax.ShapeDtypeStruct((N_MOLS, 1, HIDDEN), F32),
    )(h3, z3, wb, bb, sz)


def _mm_body(x_ref, w_ref, o_ref):
    o_ref[...] = jnp.dot(x_ref[...], w_ref[...], preferred_element_type=F32)


@jax.jit
def _k_mm(x, w):
    n = NAP // RB
    return pl.pallas_call(
        _mm_body,
        grid=(n,),
        in_specs=[_row_spec(NAP), _w_spec()],
        out_specs=_row_spec(NAP),
        out_shape=jax.ShapeDtypeStruct((NAP, HIDDEN), F32),
    )(x, w)


def kernel(f_atoms, f_bonds, a2b, b2a, b2revb, a_scope,
           W_i, W_h, W_o, b_o, W_a, W_b, b_b):
    whn = -W_h
    a2bt = (jnp.pad(a2b, ((0, NAP - N_ATOMS), (0, 0)))
            .reshape(NW, NCHUNK_A, CH, MAX_NB).transpose(0, 1, 3, 2)
            .reshape(-1))

    inputs, pn = _k_in(f_bonds, W_i, whn)
    for t in range(DEPTH - 1):
        ap = _g0(pn, a2bt, negate=True)
        q = _g1(ap, pn, b2a, b2revb)
        if t == DEPTH - 2:
            m = _k_fuse_last(q, inputs)
        else:
            pn = _k_fuse(q, inputs, whn)
    amsg = _g0(m, a2bt)

    fa_p = jnp.pad(f_atoms, ((0, NAP - N_ATOMS), (0, 0)))
    ah = _k_atom(fa_p, amsg, W_o, b_o.reshape(1, HIDDEN))
    t_all = _k_mm(ah, W_a)
    h3 = ah[:N_ATOMS].reshape(N_MOLS, MOL_SIZE, HIDDEN)
    t3 = t_all[:N_ATOMS].reshape(N_MOLS, MOL_SIZE, HIDDEN)
    z3 = _k_att_core(h3, t3)
    sz = a_scope[:, 1].astype(F32).reshape(N_MOLS, 1, 1)
    out = _k_att_out(h3, z3, W_b, b_b.reshape(1, HIDDEN), sz)
    return out.reshape(N_MOLS, HIDDEN)
